# baseline (speedup 1.0000x reference)
import functools
import jax
import jax.numpy as jnp
from jax import lax
from jax.experimental import pallas as pl
from jax.experimental.pallas import tpu as pltpu
from jax.experimental.pallas import tpu_sc as plsc

NC, NS, L = 2, 16, 16
NW = NC * NS
N, D = 16384, 128
RPW = N // NW
NV = D // L


def _perm(v, idx):
    return lax.gather(
        v,
        idx[:, None],
        lax.GatherDimensionNumbers(
            offset_dims=(), collapsed_slice_dims=(0,), start_index_map=(0,)
        ),
        slice_sizes=(1,),
        mode=lax.GatherScatterMode.PROMISE_IN_BOUNDS,
    )


def _comb(a, b, idx, m):
    return jnp.where(m, a + _perm(a, idx), b + _perm(b, idx))


HALF = RPW // 2


def _rownorm_body(z_hbm, out_hbm, zb0, zb1, obuf, s0, s1):
    wid = lax.axis_index("c") * NS + lax.axis_index("s")
    base = wid * RPW
    cp0 = pltpu.async_copy(z_hbm.at[pl.ds(base, HALF)], zb0, s0)
    cp1 = pltpu.async_copy(z_hbm.at[pl.ds(base + HALF, HALF)], zb1, s1)

    lane = lax.iota(jnp.int32, L)
    idx8 = (lane + 8) & 15
    idx4 = (lane & 8) | ((lane + 4) & 7)
    idx2 = (lane & 12) | ((lane + 2) & 3)
    idx1 = lane ^ 1
    m8 = lane < 8
    m4 = (lane & 4) == 0
    m2 = (lane & 2) == 0
    m1 = (lane & 1) == 0

    def make_group(zbuf, obase):
        def group(g):
            row0 = g * L
            vs = []
            for r in range(L):
                acc = None
                for c in range(NV):
                    z = zbuf[row0 + r, pl.ds(c * L, L)]
                    sq = z * z
                    acc = sq if acc is None else acc + sq
                vs.append(acc)
            w = [_comb(vs[i], vs[i + 8], idx8, m8) for i in range(8)]
            w = [_comb(w[i], w[i + 4], idx4, m4) for i in range(4)]
            w = [_comb(w[i], w[i + 2], idx2, m2) for i in range(2)]
            final = _comb(w[0], w[1], idx1, m1)
            obuf[pl.ds(obase + row0, L)] = final * -0.5

        return group

    cp0.wait()
    plsc.parallel_loop(0, HALF // L, 1)(make_group(zb0, 0))
    cp1.wait()
    plsc.parallel_loop(0, HALF // L, 1)(make_group(zb1, HALF))

    pltpu.sync_copy(obuf, out_hbm.at[pl.ds(base, RPW)])


@functools.partial(
    pl.kernel,
    out_type=jax.ShapeDtypeStruct((N,), jnp.float32),
    mesh=plsc.VectorSubcoreMesh(core_axis_name="c", subcore_axis_name="s"),
    scratch_types=[
        pltpu.VMEM((HALF, D), jnp.float32),
        pltpu.VMEM((HALF, D), jnp.float32),
        pltpu.VMEM((RPW,), jnp.float32),
        pltpu.SemaphoreType.DMA,
        pltpu.SemaphoreType.DMA,
    ],
)
def _rownorm(z_hbm, out_hbm, zb0, zb1, obuf, s0, s1):
    _rownorm_body(z_hbm, out_hbm, zb0, zb1, obuf, s0, s1)


def kernel(Z, labels):
    del labels
    return _rownorm(Z)


# speedup vs baseline: 1.0332x; 1.0332x over previous
import functools
import jax
import jax.numpy as jnp
from jax import lax
from jax.experimental import pallas as pl
from jax.experimental.pallas import tpu as pltpu
from jax.experimental.pallas import tpu_sc as plsc

NC, NS, L = 2, 16, 16
NW = NC * NS
N, D = 16384, 128
RPW = N // NW
NV = D // L


def _perm(v, idx):
    return lax.gather(
        v,
        idx[:, None],
        lax.GatherDimensionNumbers(
            offset_dims=(), collapsed_slice_dims=(0,), start_index_map=(0,)
        ),
        slice_sizes=(1,),
        mode=lax.GatherScatterMode.PROMISE_IN_BOUNDS,
    )


def _comb(a, b, idx, m):
    return jnp.where(m, a + _perm(a, idx), b + _perm(b, idx))


def _rownorm_body(z_hbm, out_hbm, zbuf, obuf):
    wid = lax.axis_index("c") * NS + lax.axis_index("s")
    base = wid * RPW
    pltpu.sync_copy(z_hbm.at[pl.ds(base, RPW)], zbuf)

    lane = lax.iota(jnp.int32, L)
    idx8 = (lane + 8) & 15
    idx4 = (lane & 8) | ((lane + 4) & 7)
    idx2 = (lane & 12) | ((lane + 2) & 3)
    idx1 = lane ^ 1
    m8 = lane < 8
    m4 = (lane & 4) == 0
    m2 = (lane & 2) == 0
    m1 = (lane & 1) == 0

    @plsc.parallel_loop(0, RPW // L, 1)
    def group(g):
        row0 = g * L
        vs = []
        for r in range(L):
            acc = None
            for c in range(NV):
                z = zbuf[row0 + r, pl.ds(c * L, L)]
                sq = z * z
                acc = sq if acc is None else acc + sq
            vs.append(acc)
        w = [_comb(vs[i], vs[i + 8], idx8, m8) for i in range(8)]
        w = [_comb(w[i], w[i + 4], idx4, m4) for i in range(4)]
        w = [_comb(w[i], w[i + 2], idx2, m2) for i in range(2)]
        final = _comb(w[0], w[1], idx1, m1)
        obuf[pl.ds(row0, L)] = final * -0.5

    pltpu.sync_copy(obuf, out_hbm.at[pl.ds(base, RPW)])


@functools.partial(
    pl.kernel,
    out_type=jax.ShapeDtypeStruct((N,), jnp.float32),
    mesh=plsc.VectorSubcoreMesh(core_axis_name="c", subcore_axis_name="s"),
    scratch_types=[
        pltpu.VMEM((RPW, D), jnp.float32),
        pltpu.VMEM((RPW,), jnp.float32),
    ],
)
def _rownorm(z_hbm, out_hbm, zbuf, obuf):
    _rownorm_body(z_hbm, out_hbm, zbuf, obuf)


def kernel(Z, labels):
    del labels
    return _rownorm(Z)
